# Initial kernel scaffold; baseline (speedup 1.0000x reference)
import jax
import jax.numpy as jnp
from jax.experimental import pallas as pl


def kernel(x, edge_index, W1, b1, W2, b2, gamma, beta, W_out, b_out):
    raise NotImplementedError("write your pallas kernel here")


# speedup vs baseline: 7.6340x; 7.6340x over previous
import functools

import jax
import jax.numpy as jnp
from jax import lax
from jax.experimental import pallas as pl
from jax.experimental.pallas import tpu as pltpu
from jax.experimental.pallas import tpu_sc as plsc

N = 10000
F = 128
NC = 2
NS = 16
NW = NC * NS
CHUNK = 128
N_PAD = 10240
RPT = N_PAD // NS
RB = 1000
GRID = N // RB

_E = 320000
CPW = -(-_E // (NW * CHUNK))
E_PAD = NW * CPW * CHUNK

_mesh = plsc.VectorSubcoreMesh(core_axis_name="c", subcore_axis_name="s")


def _deg_body(dst_hbm, out_hbm, dstc_v, ones_v, acc_sh):
    c = lax.axis_index("c")
    s = lax.axis_index("s")
    w = c * NS + s

    def _fill(i, val):
        for j in range(8):
            ones_v[i, pl.ds(j * 16, 16)] = jnp.full((16,), val, jnp.float32)

    lax.fori_loop(0, CHUNK, lambda i, _: (_fill(i, 0.0), 0)[1], 0)
    base = s * RPT
    for k in range(RPT // CHUNK):
        pltpu.sync_copy(ones_v, acc_sh.at[pl.ds(base + k * CHUNK, CHUNK)])
    plsc.subcore_barrier()

    lax.fori_loop(0, CHUNK, lambda i, _: (_fill(i, 1.0), 0)[1], 0)

    def body(j, _):
        pltpu.sync_copy(dst_hbm.at[w, j], dstc_v)
        pltpu.sync_copy(ones_v, acc_sh.at[dstc_v], add=True)
        return 0

    lax.fori_loop(0, CPW, body, 0)
    plsc.subcore_barrier()
    pltpu.sync_copy(acc_sh.at[pl.ds(base, RPT)], out_hbm.at[c, pl.ds(base, RPT)])


def _sc_degrees(dst3):
    return pl.kernel(
        _deg_body,
        out_type=jax.ShapeDtypeStruct((NC, N_PAD, F), jnp.float32),
        mesh=_mesh,
        scratch_types=[
            pltpu.VMEM((CHUNK,), jnp.int32),
            pltpu.VMEM((CHUNK, F), jnp.float32),
            pltpu.VMEM_SHARED((N_PAD, F), jnp.float32),
        ],
    )(dst3)


def _agg_body(y_hbm, src_hbm, dst_hbm, out_hbm,
              srcc_v, dstc_v, rows_v, acc_sh, sem):
    c = lax.axis_index("c")
    s = lax.axis_index("s")
    w = c * NS + s

    def zrow(i, _):
        for j in range(8):
            rows_v[i, pl.ds(j * 16, 16)] = jnp.zeros((16,), jnp.float32)
        return 0

    lax.fori_loop(0, CHUNK, zrow, 0)
    base = s * RPT
    for k in range(RPT // CHUNK):
        pltpu.sync_copy(rows_v, acc_sh.at[pl.ds(base + k * CHUNK, CHUNK)])
    plsc.subcore_barrier()

    def body(j, _):
        pltpu.sync_copy(src_hbm.at[w, j], srcc_v)
        pltpu.sync_copy(dst_hbm.at[w, j], dstc_v)
        pltpu.async_copy(y_hbm.at[srcc_v], rows_v, sem).wait()
        pltpu.sync_copy(rows_v, acc_sh.at[dstc_v], add=True)
        return 0

    lax.fori_loop(0, CPW, body, 0)
    plsc.subcore_barrier()
    pltpu.sync_copy(acc_sh.at[pl.ds(base, RPT)], out_hbm.at[c, pl.ds(base, RPT)])


def _sc_aggregate(y, src3, dst3):
    return pl.kernel(
        _agg_body,
        out_type=jax.ShapeDtypeStruct((NC, N_PAD, F), jnp.float32),
        mesh=_mesh,
        scratch_types=[
            pltpu.VMEM((CHUNK,), jnp.int32),
            pltpu.VMEM((CHUNK,), jnp.int32),
            pltpu.VMEM((CHUNK, F), jnp.float32),
            pltpu.VMEM_SHARED((N_PAD, F), jnp.float32),
            pltpu.SemaphoreType.DMA,
        ],
    )(y, src3, dst3)


def _dis_body(d0_ref, d1_ref, o_ref):
    o_ref[...] = lax.rsqrt(1.0 + d0_ref[...][:, :1] + d1_ref[...][:, :1])


def _tc_dis(d0, d1):
    return pl.pallas_call(
        _dis_body,
        grid=(GRID,),
        in_specs=[pl.BlockSpec((RB, F), lambda i: (i, 0)),
                  pl.BlockSpec((RB, F), lambda i: (i, 0))],
        out_specs=pl.BlockSpec((RB, 1), lambda i: (i, 0)),
        out_shape=jax.ShapeDtypeStruct((N, 1), jnp.float32),
    )(d0, d1)


def _mm_body(h_ref, w_ref, dis_ref, y_ref):
    y_ref[...] = dis_ref[...] * jnp.dot(
        h_ref[...], w_ref[...], preferred_element_type=jnp.float32)


def _tc_matmul_scaled(h, W, dis):
    fin = h.shape[1]
    return pl.pallas_call(
        _mm_body,
        grid=(GRID,),
        in_specs=[pl.BlockSpec((RB, fin), lambda i: (i, 0)),
                  pl.BlockSpec((fin, F), lambda i: (0, 0)),
                  pl.BlockSpec((RB, 1), lambda i: (i, 0))],
        out_specs=pl.BlockSpec((RB, F), lambda i: (i, 0)),
        out_shape=jax.ShapeDtypeStruct((N, F), jnp.float32),
    )(h, W, dis)


def _post_body(p0_ref, p1_ref, y_ref, dis_ref, b_ref, t_ref, st_ref):
    i = pl.program_id(0)
    pre = dis_ref[...] * (p0_ref[...] + p1_ref[...] + y_ref[...]) + b_ref[...]
    t = jnp.maximum(pre, 0.0)
    t_ref[...] = t
    ssum = jnp.sum(t, axis=0, keepdims=True)
    ssq = jnp.sum(t * t, axis=0, keepdims=True)
    st = jnp.concatenate([ssum, ssq, jnp.zeros((6, F), jnp.float32)], axis=0)

    @pl.when(i == 0)
    def _():
        st_ref[...] = st

    @pl.when(i != 0)
    def _():
        st_ref[...] += st


def _tc_post(p0, p1, y, dis, b):
    return pl.pallas_call(
        _post_body,
        grid=(GRID,),
        in_specs=[pl.BlockSpec((RB, F), lambda i: (i, 0)),
                  pl.BlockSpec((RB, F), lambda i: (i, 0)),
                  pl.BlockSpec((RB, F), lambda i: (i, 0)),
                  pl.BlockSpec((RB, 1), lambda i: (i, 0)),
                  pl.BlockSpec((1, F), lambda i: (0, 0))],
        out_specs=[pl.BlockSpec((RB, F), lambda i: (i, 0)),
                   pl.BlockSpec((8, F), lambda i: (0, 0))],
        out_shape=[jax.ShapeDtypeStruct((N, F), jnp.float32),
                   jax.ShapeDtypeStruct((8, F), jnp.float32)],
    )(p0, p1, y, dis, b)


def _bn_body(t_ref, st_ref, g_ref, bt_ref, h_ref):
    inv_n = 1.0 / N
    mean = st_ref[0:1, :] * inv_n
    var = st_ref[1:2, :] * inv_n - mean * mean
    istd = lax.rsqrt(var + 1e-5)
    h_ref[...] = (t_ref[...] - mean) * istd * g_ref[...] + bt_ref[...]


def _tc_bn(t, st, g, bt):
    return pl.pallas_call(
        _bn_body,
        grid=(GRID,),
        in_specs=[pl.BlockSpec((RB, F), lambda i: (i, 0)),
                  pl.BlockSpec((8, F), lambda i: (0, 0)),
                  pl.BlockSpec((1, F), lambda i: (0, 0)),
                  pl.BlockSpec((1, F), lambda i: (0, 0))],
        out_specs=pl.BlockSpec((RB, F), lambda i: (i, 0)),
        out_shape=jax.ShapeDtypeStruct((N, F), jnp.float32),
    )(t, st, g, bt)


def _final_body(p0_ref, p1_ref, y_ref, dis_ref, b_ref, o_ref):
    pre = dis_ref[...] * (p0_ref[...] + p1_ref[...] + y_ref[...]) + b_ref[...]
    o_ref[...] = jnp.maximum(pre, 0.0)


def _tc_final(p0, p1, y, dis, b):
    return pl.pallas_call(
        _final_body,
        grid=(GRID,),
        in_specs=[pl.BlockSpec((RB, F), lambda i: (i, 0)),
                  pl.BlockSpec((RB, F), lambda i: (i, 0)),
                  pl.BlockSpec((RB, F), lambda i: (i, 0)),
                  pl.BlockSpec((RB, 1), lambda i: (i, 0)),
                  pl.BlockSpec((1, F), lambda i: (0, 0))],
        out_specs=pl.BlockSpec((RB, F), lambda i: (i, 0)),
        out_shape=jax.ShapeDtypeStruct((N, F), jnp.float32),
    )(p0, p1, y, dis, b)


def kernel(x, edge_index, W1, b1, W2, b2, gamma, beta, W_out, b_out):
    src = edge_index[0].astype(jnp.int32)
    dst = edge_index[1].astype(jnp.int32)
    pad = E_PAD - src.shape[0]
    src3 = jnp.concatenate([src, jnp.zeros((pad,), jnp.int32)])
    dst3 = jnp.concatenate([dst, jnp.full((pad,), N, jnp.int32)])
    src3 = src3.reshape(NW, CPW, CHUNK)
    dst3 = dst3.reshape(NW, CPW, CHUNK)

    degp = _sc_degrees(dst3)
    dis = _tc_dis(degp[0, :N], degp[1, :N])

    b1r = b1.reshape(1, F)
    b2r = b2.reshape(1, F)
    bor = b_out.reshape(1, F)
    gr = gamma.reshape(1, F)
    btr = beta.reshape(1, F)

    y1 = _tc_matmul_scaled(x, W1, dis)
    p1 = _sc_aggregate(y1, src3, dst3)
    t1, st1 = _tc_post(p1[0, :N], p1[1, :N], y1, dis, b1r)
    h1 = _tc_bn(t1, st1, gr, btr)

    y2 = _tc_matmul_scaled(h1, W2, dis)
    p2 = _sc_aggregate(y2, src3, dst3)
    t2, st2 = _tc_post(p2[0, :N], p2[1, :N], y2, dis, b2r)
    h2 = _tc_bn(t2, st2, gr, btr)

    hcat = jnp.concatenate([x, h1, h2], axis=1)
    y3 = _tc_matmul_scaled(hcat, W_out, dis)
    p3 = _sc_aggregate(y3, src3, dst3)
    return _tc_final(p3[0, :N], p3[1, :N], y3, dis, bor)
